# baseline (speedup 1.0000x reference)
import functools

import jax
import jax.numpy as jnp
from jax import lax
from jax.experimental import pallas as pl
from jax.experimental.pallas import tpu as pltpu
from jax.experimental.pallas import tpu_sc as plsc

B = 64
CAP = 2048
SEQ = 1024
D = 256
NC, NS = 2, 16
NW = NC * NS
BPW = B // NW
CHUNK = 64
LANES = 16
DEPTH = 6

_mesh = plsc.VectorSubcoreMesh(
    core_axis_name="c", subcore_axis_name="s", num_cores=NC, num_subcores=NS
)


@functools.partial(
    pl.kernel,
    out_type=jax.ShapeDtypeStruct((B * CAP, D), jnp.float32),
    mesh=_mesh,
    scratch_types=(
        [pltpu.VMEM((B + LANES,), jnp.int32)]
        + [pltpu.VMEM((CHUNK,), jnp.int32) for _ in range(DEPTH)]
        + [pltpu.VMEM((CHUNK, D), jnp.float32) for _ in range(DEPTH)]
        + [pltpu.SemaphoreType.DMA for _ in range(2 * DEPTH)]
    ),
)
def _scatter(buf_hbm, obs_hbm, idx_hbm, out_hbm, idx_v, *scratch):
    idx_refs = scratch[:DEPTH]
    data_refs = scratch[DEPTH:2 * DEPTH]
    sems_in = scratch[2 * DEPTH:3 * DEPTH]
    sems_out = scratch[3 * DEPTH:4 * DEPTH]

    wid = lax.axis_index("s") * NC + lax.axis_index("c")
    pltpu.sync_copy(idx_hbm, idx_v.at[pl.ds(0, B)])
    lanes = lax.iota(jnp.int32, LANES)

    def wait_in(s):
        pltpu.make_async_copy(
            obs_hbm.at[pl.ds(0, CHUNK)], data_refs[s], sems_in[s]).wait()

    def wait_out(s):
        pltpu.make_async_copy(
            data_refs[s], out_hbm.at[pl.ds(0, CHUNK)], sems_out[s]).wait()

    def fill_ids(s, base, circ0):
        for v in range(CHUNK // LANES):
            off = circ0 + v * LANES + lanes
            idx_refs[s][pl.ds(v * LANES, LANES)] = base + (off & (CAP - 1))

    jobs = []
    for k in range(BPW):
        b = wid * BPW + k
        i = idx_v[pl.ds(b, LANES)][0]

        for c in range(SEQ // CHUNK):
            def start_in(s, b=b, c=c):
                pltpu.async_copy(
                    obs_hbm.at[pl.ds(
                        pl.multiple_of(b * SEQ + c * CHUNK, CHUNK), CHUNK)],
                    data_refs[s], sems_in[s])

            def start_out(s, b=b, i=i, c=c):
                fill_ids(s, b * CAP, i + c * CHUNK)
                pltpu.async_copy(data_refs[s], out_hbm.at[idx_refs[s]],
                                 sems_out[s])

            jobs.append((None, start_in, start_out))

        for c in range(CAP // CHUNK):
            s_c = (c * CHUNK - i) & (CAP - 1)
            is_buf = jnp.logical_and(s_c >= SEQ, s_c <= CAP - CHUNK)

            def start_in(s, b=b, c=c):
                pltpu.async_copy(
                    buf_hbm.at[pl.ds(
                        pl.multiple_of(b * CAP + c * CHUNK, CHUNK), CHUNK)],
                    data_refs[s], sems_in[s])

            def start_out(s, b=b, c=c):
                pltpu.async_copy(
                    data_refs[s],
                    out_hbm.at[pl.ds(
                        pl.multiple_of(b * CAP + c * CHUNK, CHUNK), CHUNK)],
                    sems_out[s])

            jobs.append((is_buf, start_in, start_out))

        for which in range(2):
            circ0 = i - CHUNK if which == 0 else i + SEQ

            def start_in(s, b=b, circ0=circ0):
                fill_ids(s, b * CAP, circ0)
                pltpu.async_copy(buf_hbm.at[idx_refs[s]], data_refs[s],
                                 sems_in[s])

            def start_out(s):
                pltpu.async_copy(data_refs[s], out_hbm.at[idx_refs[s]],
                                 sems_out[s])

            jobs.append((None, start_in, start_out))

    def guarded(cond, fn, *args):
        if cond is None:
            fn(*args)
        else:
            pl.when(cond)(lambda: fn(*args))

    slots = {}
    out_pending = []
    in_pending = []
    for j, (cond, start_in, start_out) in enumerate(jobs):
        s = j % DEPTH
        if j >= DEPTH:
            while out_pending and out_pending[0] <= j - DEPTH:
                jj = out_pending.pop(0)
                guarded(jobs[jj][0], wait_out, slots[jj])
        slots[j] = s
        guarded(cond, start_in, s)
        in_pending.append(j)
        while len(in_pending) > 3:
            jj = in_pending.pop(0)
            guarded(jobs[jj][0], wait_in, slots[jj])
            guarded(jobs[jj][0], jobs[jj][2], slots[jj])
            out_pending.append(jj)

    while in_pending:
        jj = in_pending.pop(0)
        guarded(jobs[jj][0], wait_in, slots[jj])
        guarded(jobs[jj][0], jobs[jj][2], slots[jj])
        out_pending.append(jj)
    while out_pending:
        jj = out_pending.pop(0)
        guarded(jobs[jj][0], wait_out, slots[jj])


def kernel(buffer, observation_sequence, index, size):
    del size
    buf2d = buffer.reshape(B * CAP, D)
    obs2d = observation_sequence.reshape(B * SEQ, D)
    out2d = _scatter(buf2d, obs2d, index)
    return out2d.reshape(B, CAP, D)


# speedup vs baseline: 1.0085x; 1.0085x over previous
import functools

import jax
import jax.numpy as jnp
from jax import lax
from jax.experimental import pallas as pl
from jax.experimental.pallas import tpu as pltpu
from jax.experimental.pallas import tpu_sc as plsc

B = 64
CAP = 2048
SEQ = 1024
D = 256
NC, NS = 2, 16
NW = NC * NS
BPW = B // NW
CHUNK = 128
LANES = 16
DEPTH = 3

_mesh = plsc.VectorSubcoreMesh(
    core_axis_name="c", subcore_axis_name="s", num_cores=NC, num_subcores=NS
)


@functools.partial(
    pl.kernel,
    out_type=jax.ShapeDtypeStruct((B * CAP, D), jnp.float32),
    mesh=_mesh,
    scratch_types=(
        [pltpu.VMEM((B + LANES,), jnp.int32)]
        + [pltpu.VMEM((CHUNK,), jnp.int32) for _ in range(DEPTH)]
        + [pltpu.VMEM((CHUNK, D), jnp.float32) for _ in range(DEPTH)]
        + [pltpu.SemaphoreType.DMA for _ in range(2 * DEPTH)]
    ),
)
def _scatter(buf_hbm, obs_hbm, idx_hbm, out_hbm, idx_v, *scratch):
    idx_refs = scratch[:DEPTH]
    data_refs = scratch[DEPTH:2 * DEPTH]
    sems_in = scratch[2 * DEPTH:3 * DEPTH]
    sems_out = scratch[3 * DEPTH:4 * DEPTH]

    wid = lax.axis_index("s") * NC + lax.axis_index("c")
    pltpu.sync_copy(idx_hbm, idx_v.at[pl.ds(0, B)])
    lanes = lax.iota(jnp.int32, LANES)

    def wait_in(s):
        pltpu.make_async_copy(
            obs_hbm.at[pl.ds(0, CHUNK)], data_refs[s], sems_in[s]).wait()

    def wait_out(s):
        pltpu.make_async_copy(
            data_refs[s], out_hbm.at[pl.ds(0, CHUNK)], sems_out[s]).wait()

    def fill_ids(s, b, circ0):
        for v in range(CHUNK // LANES):
            off = circ0 + v * LANES + lanes
            idx_refs[s][pl.ds(v * LANES, LANES)] = b * CAP + (off & (CAP - 1))

    jobs = []
    for k in range(BPW):
        b = wid * BPW + k
        i = idx_v[pl.ds(b, LANES)][0]

        for c in range(SEQ // CHUNK):
            def start_in(s, b=b, c=c):
                pltpu.async_copy(
                    obs_hbm.at[pl.ds(b * SEQ + c * CHUNK, CHUNK)],
                    data_refs[s], sems_in[s])

            def start_out(s, b=b, i=i, c=c):
                fill_ids(s, b, i + c * CHUNK)
                pltpu.async_copy(data_refs[s], out_hbm.at[idx_refs[s]],
                                 sems_out[s])

            jobs.append((start_in, start_out))

        for c in range(SEQ // CHUNK):
            def start_in(s, b=b, i=i, c=c):
                fill_ids(s, b, i + SEQ + c * CHUNK)
                pltpu.async_copy(buf_hbm.at[idx_refs[s]], data_refs[s],
                                 sems_in[s])

            def start_out(s):
                pltpu.async_copy(data_refs[s], out_hbm.at[idx_refs[s]],
                                 sems_out[s])

            jobs.append((start_in, start_out))

    slots = {}
    out_pending = []
    in_pending = []
    for j, (start_in, start_out) in enumerate(jobs):
        s = j % DEPTH
        if j >= DEPTH:
            while out_pending and out_pending[0] <= j - DEPTH:
                wait_out(slots[out_pending.pop(0)])
        slots[j] = s
        start_in(s)
        in_pending.append(j)
        while len(in_pending) > 1:
            jj = in_pending.pop(0)
            wait_in(slots[jj])
            jobs[jj][1](slots[jj])
            out_pending.append(jj)

    while in_pending:
        jj = in_pending.pop(0)
        wait_in(slots[jj])
        jobs[jj][1](slots[jj])
        out_pending.append(jj)
    while out_pending:
        wait_out(slots[out_pending.pop(0)])


def kernel(buffer, observation_sequence, index, size):
    del size
    buf2d = buffer.reshape(B * CAP, D)
    obs2d = observation_sequence.reshape(B * SEQ, D)
    out2d = _scatter(buf2d, obs2d, index)
    return out2d.reshape(B, CAP, D)
